# baseline (speedup 1.0000x reference)
import functools

import jax
import jax.numpy as jnp
from jax import lax
from jax.experimental import pallas as pl
from jax.experimental.pallas import tpu as pltpu
from jax.experimental.pallas import tpu_sc as plsc

N = 10000
E = 160000
D_IN = 256
HIDDEN = 512
NCLS = 64

NCORES = 2
NSUB = 16
NW = NCORES * NSUB
CH = 125
CPW = E // (NW * CH)
SEG = N // NSUB
ROWB = 1000
NPAD = 10240
PREPS = NPAD // NW
NROWS2D = E // CH

_MESH = plsc.VectorSubcoreMesh(core_axis_name="c", subcore_axis_name="s")
_SC_PARAMS = pltpu.CompilerParams(use_tc_tiling_on_sc=False,
                                  needs_layout_passes=False)


def _mlp_body(x_ref, w1_ref, b1_ref, w2_ref, b2_ref, xmid_ref, h_ref):
    xm = jnp.dot(x_ref[...], w1_ref[...], preferred_element_type=jnp.float32)
    xm = jnp.maximum(xm + b1_ref[...], 0.0)
    xmid_ref[...] = xm
    h_ref[...] = (
        jnp.dot(xm, w2_ref[...], preferred_element_type=jnp.float32) + b2_ref[...]
    )


def _mlp(x, W1, b1, W2, b2):
    return pl.pallas_call(
        _mlp_body,
        grid=(N // ROWB,),
        in_specs=[
            pl.BlockSpec((ROWB, D_IN), lambda i: (i, 0)),
            pl.BlockSpec((D_IN, HIDDEN), lambda i: (0, 0)),
            pl.BlockSpec((1, HIDDEN), lambda i: (0, 0)),
            pl.BlockSpec((HIDDEN, NCLS), lambda i: (0, 0)),
            pl.BlockSpec((1, NCLS), lambda i: (0, 0)),
        ],
        out_specs=[
            pl.BlockSpec((ROWB, HIDDEN), lambda i: (i, 0)),
            pl.BlockSpec((ROWB, NCLS), lambda i: (i, 0)),
        ],
        out_shape=[
            jax.ShapeDtypeStruct((N, HIDDEN), jnp.float32),
            jax.ShapeDtypeStruct((NPAD, NCLS), jnp.float32),
        ],
    )(x, W1, b1, W2, b2)


def _mid_body(u1p_ref, dexp_ref, u1s_ref, g1_ref):
    u1s = u1p_ref[0] + u1p_ref[1]
    u1s_ref[...] = u1s
    dv = dexp_ref[...]
    g1_ref[...] = (dv * dv) * u1s


def _mid(u1p, dexp):
    return pl.pallas_call(
        _mid_body,
        grid=(N // ROWB,),
        in_specs=[
            pl.BlockSpec((2, ROWB, NCLS), lambda i: (0, i, 0)),
            pl.BlockSpec((ROWB, NCLS), lambda i: (i, 0)),
        ],
        out_specs=[
            pl.BlockSpec((ROWB, NCLS), lambda i: (i, 0)),
            pl.BlockSpec((ROWB, NCLS), lambda i: (i, 0)),
        ],
        out_shape=[
            jax.ShapeDtypeStruct((N, NCLS), jnp.float32),
            jax.ShapeDtypeStruct((NPAD, NCLS), jnp.float32),
        ],
    )(u1p, dexp)


def _comb_body(temp_ref, h_ref, u1s_ref, u2p_ref, dexp_ref, out_ref):
    t0 = jnp.maximum(temp_ref[0], 0.0)
    t1 = jnp.maximum(temp_ref[1], 0.0)
    t2 = jnp.maximum(temp_ref[2], 0.0)
    c0 = (t0 + 2.0 * t1 + t2) * 0.25
    c1 = (t0 - t2) * 0.5
    c2 = (t0 - 2.0 * t1 + t2) * 0.25
    dv = dexp_ref[...]
    u2s = u2p_ref[0] + u2p_ref[1]
    out_ref[...] = c0 * h_ref[...] + dv * (c1 * u1s_ref[...] + c2 * u2s)


def _comb(temp, h, u1s, u2p, dexp):
    return pl.pallas_call(
        _comb_body,
        grid=(N // ROWB,),
        in_specs=[
            pl.BlockSpec(memory_space=pltpu.SMEM),
            pl.BlockSpec((ROWB, NCLS), lambda i: (i, 0)),
            pl.BlockSpec((ROWB, NCLS), lambda i: (i, 0)),
            pl.BlockSpec((2, ROWB, NCLS), lambda i: (0, i, 0)),
            pl.BlockSpec((ROWB, NCLS), lambda i: (i, 0)),
        ],
        out_specs=pl.BlockSpec((ROWB, NCLS), lambda i: (i, 0)),
        out_shape=jax.ShapeDtypeStruct((N, NCLS), jnp.float32),
    )(temp, h, u1s, u2p, dexp)


def _prep_body(src_hbm, h_hbm, p_hbm, dexp_hbm,
               idx_v, ones_v, zbuf_v, deg_v, dinv_v, hrow_v, out_v, acc_sh):
    c = lax.axis_index("c")
    s = lax.axis_index("s")
    one = jnp.full((16,), 1.0, jnp.float32)
    zero = jnp.zeros((16,), jnp.float32)
    for k in range(8):
        ones_v[pl.ds(k * 16, 16)] = one
    for k in range(40):
        zbuf_v[pl.ds(k * 16, 16)] = zero
    pltpu.sync_copy(zbuf_v, acc_sh.at[pl.ds(s * 640, 640)])
    pltpu.sync_copy(src_hbm.at[pl.ds(s * (NROWS2D // NSUB), NROWS2D // NSUB)],
                    idx_v)
    plsc.subcore_barrier()

    def body(j, carry):
        pltpu.sync_copy(ones_v.at[pl.ds(0, CH)], acc_sh.at[idx_v.at[j]], add=True)
        return carry

    lax.fori_loop(0, NROWS2D // NSUB, body, 0)
    plsc.subcore_barrier()

    w = c * NSUB + s
    r0 = w * PREPS
    pltpu.sync_copy(acc_sh.at[pl.ds(r0, PREPS)], deg_v)

    def nr(k, carry):
        x = deg_v[pl.ds(k * 16, 16)]
        i = lax.bitcast_convert_type(x, jnp.int32)
        i = 0x5F3759DF - lax.shift_right_arithmetic(i, 1)
        y = lax.bitcast_convert_type(i, jnp.float32)
        for _ in range(4):
            y = y * (1.5 - 0.5 * x * y * y)
        dinv_v[pl.ds(k * 16, 16)] = jnp.where(x > 0.0, y, 0.0)
        return carry

    lax.fori_loop(0, PREPS // 16, nr, 0)
    pltpu.sync_copy(h_hbm.at[pl.ds(r0, PREPS)], hrow_v)

    def rowf(r, carry):
        ridx = lax.broadcast_in_dim(r, (16,), ())
        dv = plsc.load_gather(dinv_v, [ridx])
        for k in range(NCLS // 16):
            out_v[r, pl.ds(k * 16, 16)] = dv * hrow_v[r, pl.ds(k * 16, 16)]
        for k in range(NCLS // 16):
            hrow_v[r, pl.ds(k * 16, 16)] = dv
        return carry

    lax.fori_loop(0, PREPS, rowf, 0)
    pltpu.sync_copy(out_v, p_hbm.at[pl.ds(r0, PREPS)])
    pltpu.sync_copy(hrow_v, dexp_hbm.at[pl.ds(r0, PREPS)])


def _prep(src2d, h_pad):
    f = functools.partial(
        pl.kernel,
        out_type=(
            jax.ShapeDtypeStruct((NPAD, NCLS), jnp.float32),
            jax.ShapeDtypeStruct((NPAD, NCLS), jnp.float32),
        ),
        mesh=_MESH,
        scratch_types=[
            pltpu.VMEM((NROWS2D // NSUB, CH), jnp.int32),
            pltpu.VMEM((128,), jnp.float32),
            pltpu.VMEM((640,), jnp.float32),
            pltpu.VMEM((PREPS,), jnp.float32),
            pltpu.VMEM((PREPS,), jnp.float32),
            pltpu.VMEM((PREPS, NCLS), jnp.float32),
            pltpu.VMEM((PREPS, NCLS), jnp.float32),
            pltpu.VMEM_SHARED((NPAD,), jnp.float32),
        ],
        compiler_params=_SC_PARAMS,
    )(_prep_body)
    return f(src2d, h_pad)


def _spmv_body(g_hbm, src_hbm, dst_hbm, up_hbm, si_v, di_v, buf0, buf1,
               gs0, gs1, ss0, ss1, acc_ref):
    c = lax.axis_index("c")
    s = lax.axis_index("s")
    w = c * NSUB + s
    zero = jnp.zeros((16,), jnp.float32)

    def zb(r, carry):
        for k in range(NCLS // 16):
            buf0[r, pl.ds(k * 16, 16)] = zero
        return carry

    lax.fori_loop(0, CH, zb, 0)
    for k in range(SEG // CH):
        pltpu.sync_copy(buf0, acc_ref.at[pl.ds(s * SEG + k * CH, CH)])
    pltpu.sync_copy(src_hbm.at[pl.ds(w * CPW, CPW)], si_v)
    pltpu.sync_copy(dst_hbm.at[pl.ds(w * CPW, CPW)], di_v)
    plsc.subcore_barrier()

    bufs = (buf0, buf1)
    gsems = (gs0, gs1)
    ssems = (ss0, ss1)
    gd = [None] * CPW
    sd = [None] * CPW
    gd[0] = pltpu.async_copy(g_hbm.at[si_v.at[0]], buf0, gs0)
    for j in range(CPW):
        b = j % 2
        gd[j].wait()
        if j + 1 < CPW:
            nb = (j + 1) % 2
            if j >= 1:
                sd[j - 1].wait()
            gd[j + 1] = pltpu.async_copy(
                g_hbm.at[si_v.at[j + 1]], bufs[nb], gsems[nb])
        sd[j] = pltpu.async_copy(bufs[b], acc_ref.at[di_v.at[j]], ssems[b],
                                 add=True)
    sd[CPW - 1].wait()
    sd[CPW - 2].wait()
    plsc.subcore_barrier()
    for k in range(SEG // CH):
        off = s * SEG + k * CH
        pltpu.sync_copy(acc_ref.at[pl.ds(off, CH)], buf0)
        pltpu.sync_copy(buf0, up_hbm.at[c, pl.ds(off, CH)])


def _spmv(g, src2d, dst2d):
    f = functools.partial(
        pl.kernel,
        out_type=jax.ShapeDtypeStruct((NCORES, N, NCLS), jnp.float32),
        mesh=_MESH,
        scratch_types=[
            pltpu.VMEM((CPW, CH), jnp.int32),
            pltpu.VMEM((CPW, CH), jnp.int32),
            pltpu.VMEM((CH, NCLS), jnp.float32),
            pltpu.VMEM((CH, NCLS), jnp.float32),
            pltpu.SemaphoreType.DMA,
            pltpu.SemaphoreType.DMA,
            pltpu.SemaphoreType.DMA,
            pltpu.SemaphoreType.DMA,
            pltpu.VMEM_SHARED((N, NCLS), jnp.float32),
        ],
        compiler_params=_SC_PARAMS,
    )(_spmv_body)
    return f(g, src2d, dst2d)


def kernel(edge_index, x, W1, b1, W2, b2, temp):
    src2d = edge_index[0].astype(jnp.int32).reshape(NROWS2D, CH)
    dst2d = edge_index[1].astype(jnp.int32).reshape(NROWS2D, CH)
    x_mid, h_pad = _mlp(x, W1, b1.reshape(1, HIDDEN), W2, b2.reshape(1, NCLS))
    p_pad, dexp_pad = _prep(src2d, h_pad)
    u1p = _spmv(p_pad, src2d, dst2d)
    u1s, g1_pad = _mid(u1p, dexp_pad)
    u2p = _spmv(g1_pad, src2d, dst2d)
    out = _comb(temp, h_pad, u1s, u2p, dexp_pad)
    return (out, x_mid)


# speedup vs baseline: 1.1889x; 1.1889x over previous
import functools

import jax
import jax.numpy as jnp
from jax import lax
from jax.experimental import pallas as pl
from jax.experimental.pallas import tpu as pltpu
from jax.experimental.pallas import tpu_sc as plsc

N = 10000
E = 160000
D_IN = 256
HIDDEN = 512
NCLS = 64

NCORES = 2
NSUB = 16
NW = NCORES * NSUB
CH = 125
CPW = E // (NW * CH)
SEG = N // NSUB
ROWB = 1000
NPAD = 10240
RB2 = 2000
PREPS = NPAD // NW
NROWS2D = E // CH

_MESH = plsc.VectorSubcoreMesh(core_axis_name="c", subcore_axis_name="s")
_SC_PARAMS = pltpu.CompilerParams(use_tc_tiling_on_sc=False,
                                  needs_layout_passes=False)


def _mlp_body(x_ref, w1_ref, b1_ref, w2_ref, b2_ref, xmid_ref, h_ref):
    xm = jnp.dot(x_ref[...], w1_ref[...], preferred_element_type=jnp.float32)
    xm = jnp.maximum(xm + b1_ref[...], 0.0)
    xmid_ref[...] = xm
    h_ref[...] = (
        jnp.dot(xm, w2_ref[...], preferred_element_type=jnp.float32) + b2_ref[...]
    )


def _mlp(x, W1, b1, W2, b2):
    return pl.pallas_call(
        _mlp_body,
        grid=(N // ROWB,),
        in_specs=[
            pl.BlockSpec((ROWB, D_IN), lambda i: (i, 0)),
            pl.BlockSpec((D_IN, HIDDEN), lambda i: (0, 0)),
            pl.BlockSpec((1, HIDDEN), lambda i: (0, 0)),
            pl.BlockSpec((HIDDEN, NCLS), lambda i: (0, 0)),
            pl.BlockSpec((1, NCLS), lambda i: (0, 0)),
        ],
        out_specs=[
            pl.BlockSpec((ROWB, HIDDEN), lambda i: (i, 0)),
            pl.BlockSpec((ROWB, NCLS), lambda i: (i, 0)),
        ],
        out_shape=[
            jax.ShapeDtypeStruct((N, HIDDEN), jnp.float32),
            jax.ShapeDtypeStruct((NPAD, NCLS), jnp.float32),
        ],
    )(x, W1, b1, W2, b2)


def _mid_body(u1a_ref, u1b_ref, dexp_ref, u1s_ref, g1_ref):
    u1s = u1a_ref[...] + u1b_ref[...]
    u1s_ref[...] = u1s
    dv = dexp_ref[...]
    g1_ref[...] = (dv * dv) * u1s


def _mid(u1a, u1b, dexp):
    return pl.pallas_call(
        _mid_body,
        grid=(N // RB2,),
        in_specs=[
            pl.BlockSpec((RB2, NCLS), lambda i: (i, 0)),
            pl.BlockSpec((RB2, NCLS), lambda i: (i, 0)),
            pl.BlockSpec((RB2, NCLS), lambda i: (i, 0)),
        ],
        out_specs=[
            pl.BlockSpec((RB2, NCLS), lambda i: (i, 0)),
            pl.BlockSpec((RB2, NCLS), lambda i: (i, 0)),
        ],
        out_shape=[
            jax.ShapeDtypeStruct((N, NCLS), jnp.float32),
            jax.ShapeDtypeStruct((NPAD, NCLS), jnp.float32),
        ],
    )(u1a, u1b, dexp)


def _comb_body(temp_ref, h_ref, u1s_ref, u2a_ref, u2b_ref, dexp_ref, out_ref):
    t0 = jnp.maximum(temp_ref[0], 0.0)
    t1 = jnp.maximum(temp_ref[1], 0.0)
    t2 = jnp.maximum(temp_ref[2], 0.0)
    c0 = (t0 + 2.0 * t1 + t2) * 0.25
    c1 = (t0 - t2) * 0.5
    c2 = (t0 - 2.0 * t1 + t2) * 0.25
    dv = dexp_ref[...]
    u2s = u2a_ref[...] + u2b_ref[...]
    out_ref[...] = c0 * h_ref[...] + dv * (c1 * u1s_ref[...] + c2 * u2s)


def _comb(temp, h, u1s, u2a, u2b, dexp):
    return pl.pallas_call(
        _comb_body,
        grid=(N // RB2,),
        in_specs=[
            pl.BlockSpec(memory_space=pltpu.SMEM),
            pl.BlockSpec((RB2, NCLS), lambda i: (i, 0)),
            pl.BlockSpec((RB2, NCLS), lambda i: (i, 0)),
            pl.BlockSpec((RB2, NCLS), lambda i: (i, 0)),
            pl.BlockSpec((RB2, NCLS), lambda i: (i, 0)),
            pl.BlockSpec((RB2, NCLS), lambda i: (i, 0)),
        ],
        out_specs=pl.BlockSpec((RB2, NCLS), lambda i: (i, 0)),
        out_shape=jax.ShapeDtypeStruct((N, NCLS), jnp.float32),
    )(temp, h, u1s, u2a, u2b, dexp)


def _prep_body(src_hbm, h_hbm, p_hbm, dexp_hbm,
               idx_v, ones_v, zbuf_v, deg_v, dinv_v, hrow_v, out_v, acc_sh):
    c = lax.axis_index("c")
    s = lax.axis_index("s")
    one = jnp.full((16,), 1.0, jnp.float32)
    zero = jnp.zeros((16,), jnp.float32)
    for k in range(8):
        ones_v[pl.ds(k * 16, 16)] = one
    for k in range(40):
        zbuf_v[pl.ds(k * 16, 16)] = zero
    pltpu.sync_copy(zbuf_v, acc_sh.at[pl.ds(s * 640, 640)])
    pltpu.sync_copy(src_hbm.at[pl.ds(s * (NROWS2D // NSUB), NROWS2D // NSUB)],
                    idx_v)
    plsc.subcore_barrier()

    def body(j, carry):
        pltpu.sync_copy(ones_v.at[pl.ds(0, CH)], acc_sh.at[idx_v.at[j]], add=True)
        return carry

    lax.fori_loop(0, NROWS2D // NSUB, body, 0)
    plsc.subcore_barrier()

    w = c * NSUB + s
    r0 = w * PREPS
    pltpu.sync_copy(acc_sh.at[pl.ds(r0, PREPS)], deg_v)

    def nr(k, carry):
        x = deg_v[pl.ds(k * 16, 16)]
        i = lax.bitcast_convert_type(x, jnp.int32)
        i = 0x5F3759DF - lax.shift_right_arithmetic(i, 1)
        y = lax.bitcast_convert_type(i, jnp.float32)
        for _ in range(4):
            y = y * (1.5 - 0.5 * x * y * y)
        dinv_v[pl.ds(k * 16, 16)] = jnp.where(x > 0.0, y, 0.0)
        return carry

    lax.fori_loop(0, PREPS // 16, nr, 0)
    pltpu.sync_copy(h_hbm.at[pl.ds(r0, PREPS)], hrow_v)

    def rowf(r, carry):
        ridx = lax.broadcast_in_dim(r, (16,), ())
        dv = plsc.load_gather(dinv_v, [ridx])
        for k in range(NCLS // 16):
            out_v[r, pl.ds(k * 16, 16)] = dv * hrow_v[r, pl.ds(k * 16, 16)]
        for k in range(NCLS // 16):
            hrow_v[r, pl.ds(k * 16, 16)] = dv
        return carry

    lax.fori_loop(0, PREPS, rowf, 0)
    pltpu.sync_copy(out_v, p_hbm.at[pl.ds(r0, PREPS)])
    pltpu.sync_copy(hrow_v, dexp_hbm.at[pl.ds(r0, PREPS)])


def _prep(src2d, h_pad):
    f = functools.partial(
        pl.kernel,
        out_type=(
            jax.ShapeDtypeStruct((NPAD, NCLS), jnp.float32),
            jax.ShapeDtypeStruct((NPAD, NCLS), jnp.float32),
        ),
        mesh=_MESH,
        scratch_types=[
            pltpu.VMEM((NROWS2D // NSUB, CH), jnp.int32),
            pltpu.VMEM((128,), jnp.float32),
            pltpu.VMEM((640,), jnp.float32),
            pltpu.VMEM((PREPS,), jnp.float32),
            pltpu.VMEM((PREPS,), jnp.float32),
            pltpu.VMEM((PREPS, NCLS), jnp.float32),
            pltpu.VMEM((PREPS, NCLS), jnp.float32),
            pltpu.VMEM_SHARED((NPAD,), jnp.float32),
        ],
        compiler_params=_SC_PARAMS,
    )(_prep_body)
    return f(src2d, h_pad)


NBUF = 4
LOOKAHEAD = 3


def _spmv_body(g_hbm, src_hbm, dst_hbm, ua_hbm, ub_hbm, si_v, di_v,
               buf0, buf1, buf2, buf3,
               gs0, gs1, gs2, gs3, ss0, ss1, ss2, ss3, acc_ref):
    c = lax.axis_index("c")
    s = lax.axis_index("s")
    w = c * NSUB + s
    zero = jnp.zeros((16,), jnp.float32)

    def zb(r, carry):
        for k in range(NCLS // 16):
            buf0[r, pl.ds(k * 16, 16)] = zero
        return carry

    lax.fori_loop(0, CH, zb, 0)
    for k in range(SEG // CH):
        pltpu.sync_copy(buf0, acc_ref.at[pl.ds(s * SEG + k * CH, CH)])
    pltpu.sync_copy(src_hbm.at[pl.ds(w * CPW, CPW)], si_v)
    pltpu.sync_copy(dst_hbm.at[pl.ds(w * CPW, CPW)], di_v)
    plsc.subcore_barrier()

    bufs = (buf0, buf1, buf2, buf3)
    gsems = (gs0, gs1, gs2, gs3)
    ssems = (ss0, ss1, ss2, ss3)
    gd = [None] * CPW
    sd = [None] * CPW
    for j in range(LOOKAHEAD):
        gd[j] = pltpu.async_copy(g_hbm.at[si_v.at[j]], bufs[j % NBUF],
                                 gsems[j % NBUF])
    for j in range(CPW):
        b = j % NBUF
        gd[j].wait()
        sd[j] = pltpu.async_copy(bufs[b], acc_ref.at[di_v.at[j]], ssems[b],
                                 add=True)
        nj = j + LOOKAHEAD
        if nj < CPW:
            nb = nj % NBUF
            if nj - NBUF >= 0:
                sd[nj - NBUF].wait()
            gd[nj] = pltpu.async_copy(g_hbm.at[si_v.at[nj]], bufs[nb],
                                      gsems[nb])
    for j in range(CPW - NBUF, CPW):
        if sd[j] is not None and j >= 0:
            sd[j].wait()
    plsc.subcore_barrier()

    @pl.when(c == 0)
    def _():
        for k in range(SEG // CH):
            off = s * SEG + k * CH
            pltpu.sync_copy(acc_ref.at[pl.ds(off, CH)], buf0)
            pltpu.sync_copy(buf0, ua_hbm.at[pl.ds(off, CH)])

    @pl.when(c == 1)
    def _():
        for k in range(SEG // CH):
            off = s * SEG + k * CH
            pltpu.sync_copy(acc_ref.at[pl.ds(off, CH)], buf0)
            pltpu.sync_copy(buf0, ub_hbm.at[pl.ds(off, CH)])


def _spmv(g, src2d, dst2d):
    f = functools.partial(
        pl.kernel,
        out_type=(
            jax.ShapeDtypeStruct((N, NCLS), jnp.float32),
            jax.ShapeDtypeStruct((N, NCLS), jnp.float32),
        ),
        mesh=_MESH,
        scratch_types=[
            pltpu.VMEM((CPW, CH), jnp.int32),
            pltpu.VMEM((CPW, CH), jnp.int32),
            pltpu.VMEM((CH, NCLS), jnp.float32),
            pltpu.VMEM((CH, NCLS), jnp.float32),
            pltpu.VMEM((CH, NCLS), jnp.float32),
            pltpu.VMEM((CH, NCLS), jnp.float32),
            pltpu.SemaphoreType.DMA,
            pltpu.SemaphoreType.DMA,
            pltpu.SemaphoreType.DMA,
            pltpu.SemaphoreType.DMA,
            pltpu.SemaphoreType.DMA,
            pltpu.SemaphoreType.DMA,
            pltpu.SemaphoreType.DMA,
            pltpu.SemaphoreType.DMA,
            pltpu.VMEM_SHARED((N, NCLS), jnp.float32),
        ],
        compiler_params=_SC_PARAMS,
    )(_spmv_body)
    return f(g, src2d, dst2d)


def kernel(edge_index, x, W1, b1, W2, b2, temp):
    src2d = edge_index[0].astype(jnp.int32).reshape(NROWS2D, CH)
    dst2d = edge_index[1].astype(jnp.int32).reshape(NROWS2D, CH)
    x_mid, h_pad = _mlp(x, W1, b1.reshape(1, HIDDEN), W2, b2.reshape(1, NCLS))
    p_pad, dexp_pad = _prep(src2d, h_pad)
    u1a, u1b = _spmv(p_pad, src2d, dst2d)
    u1s, g1_pad = _mid(u1a, u1b, dexp_pad)
    u2a, u2b = _spmv(g1_pad, src2d, dst2d)
    out = _comb(temp, h_pad, u1s, u2a, u2b, dexp_pad)
    return (out, x_mid)
